# baseline (speedup 1.0000x reference)
import functools

import jax
import jax.numpy as jnp
from jax import lax
from jax.experimental import pallas as pl
from jax.experimental.pallas import tpu as pltpu
from jax.experimental.pallas import tpu_sc as plsc

N = 100000
F = 9
V = 100
H = 128
HW = H // 2
L = 16
SC_R = 128
RT = 72
MS = SC_R - RT
SMS = ((MS + L - 1) // L) * L
NW = 32
NSC = (N + SC_R - 1) // SC_R


def _sc_body(x_hbm, embp_hbm, emb2_hbm, out_hbm, tables, raws, outs, idxs,
             idx_sem, tab_sem, gsems, osems):
    cid = lax.axis_index("c")
    sid = lax.axis_index("s")
    wid = sid * 2 + cid
    nj = (NSC - wid + NW - 1) // NW

    lanes = lax.broadcasted_iota(jnp.int32, (L,), 0)
    cols = [jnp.int32(cc * L) + lanes for cc in range(HW // L)]
    zeros = jnp.zeros((L,), jnp.float32)
    himask = jnp.full((L,), -65536, jnp.int32)

    def sbase(j):
        chunk = wid + j * NW
        return jnp.minimum(chunk * SC_R, N - SC_R)

    def fire_raw(j, b):
        return pltpu.async_copy(x_hbm.at[pl.ds(sbase(j), SC_R)], raws[b],
                                idx_sem)

    def wait_gathers(b):
        for _ in range(F):
            pltpu.make_async_copy(emb2_hbm.at[idxs[b].at[pl.ds(0, MS)]],
                                  outs[b].at[pl.ds(RT, MS)], gsems[b]).wait()

    def fire_out(j, b):
        return pltpu.async_copy(outs[b], out_hbm.at[pl.ds(sbase(j), SC_R)],
                                osems[b])

    def wait_out(b):
        pltpu.make_async_copy(outs[b], out_hbm.at[pl.ds(0, SC_R)],
                              osems[b]).wait()

    @pl.when(nj > 0)
    def _():
        fire_raw(0, 0)
    tcps = [pltpu.async_copy(embp_hbm.at[pl.ds(i * V * HW, V * HW)],
                             tables[i], tab_sem) for i in range(F)]
    for cp in tcps:
        cp.wait()

    def do_super(j, b):
        pltpu.make_async_copy(x_hbm.at[pl.ds(0, SC_R)], raws[b],
                              idx_sem).wait()

        @pl.when(j + 1 < nj)
        def _():
            fire_raw(j + 1, 1 - b)

        @pl.when(j >= 1)
        def _():
            wait_gathers(1 - b)
            fire_out(j - 1, 1 - b)

        @pl.when(j >= 2)
        def _():
            wait_out(b)

        for k in range(SMS // L):
            srows = jnp.minimum(jnp.int32(k * L) + lanes, MS - 1) + RT
            for i in range(F):
                vals = plsc.load_gather(raws[b],
                                        [srows, jnp.full((L,), i, jnp.int32)])
                idxs[b][pl.ds(i * SMS + k * L, L)] = vals + (i * V)

        def zrow(r, carry):
            for m in range(H // L):
                outs[b][r, pl.ds(m * L, L)] = zeros
            return carry

        lax.fori_loop(RT, SC_R, zrow, 0)

        for i in range(F):
            pltpu.async_copy(
                emb2_hbm.at[idxs[b].at[pl.ds(i * SMS, MS)]],
                outs[b].at[pl.ds(RT, MS)], gsems[b], add=True)

        def one_row(r):
            bases = []
            for i in range(F):
                bidx = plsc.load_gather(raws[b],
                                        [jnp.full((L,), r, jnp.int32),
                                         jnp.full((L,), i, jnp.int32)])
                bases.append(lax.shift_left(bidx, 6))
            for cc in range(HW // L):
                ws = [plsc.bitcast(
                          plsc.load_gather(tables[i], [bases[i] + cols[cc]]),
                          jnp.bfloat16)
                      for i in range(F)]
                while len(ws) > 1:
                    ws = [a + b2 for a, b2 in zip(ws[::2], ws[1::2])] \
                        + ([ws[-1]] if len(ws) % 2 else [])
                ai = plsc.bitcast(ws[0], jnp.int32)
                outs[b][r, pl.ds(cc * L, L)] = plsc.bitcast(
                    lax.shift_left(ai, 16), jnp.float32)
                outs[b][r, pl.ds(HW + cc * L, L)] = plsc.bitcast(
                    lax.bitwise_and(ai, himask), jnp.float32)

        def row_step(rr, carry):
            one_row(rr * 2)
            one_row(rr * 2 + 1)
            return carry

        lax.fori_loop(0, RT // 2, row_step, 0)

    def pair_step(jj, carry):
        j0 = jj * 2

        @pl.when(j0 < nj)
        def _():
            do_super(j0, 0)

        @pl.when(j0 + 1 < nj)
        def _():
            do_super(j0 + 1, 1)

        return carry

    lax.fori_loop(0, (nj + 1) // 2, pair_step, 0)

    for b in range(2):
        @pl.when((nj > 0) & ((nj - 1) % 2 == b))
        def _():
            wait_gathers(b)
            fire_out(nj - 1, b)
            wait_out(b)

        @pl.when((nj > 1) & ((nj - 2) % 2 == b))
        def _():
            wait_out(b)


@functools.lru_cache(maxsize=1)
def _build_encoder():
    @functools.partial(
        pl.kernel,
        out_type=jax.ShapeDtypeStruct((N, H), jnp.float32),
        mesh=plsc.VectorSubcoreMesh(core_axis_name="c", subcore_axis_name="s"),
        compiler_params=pltpu.CompilerParams(needs_layout_passes=False),
        scratch_types=(
            [pltpu.VMEM((V * HW,), jnp.int32) for _ in range(F)]
            + [pltpu.VMEM((SC_R, F), jnp.int32) for _ in range(2)]
            + [pltpu.VMEM((SC_R, H), jnp.float32) for _ in range(2)]
            + [pltpu.VMEM((F * SMS,), jnp.int32) for _ in range(2)]
            + [pltpu.SemaphoreType.DMA] * 6
        ),
    )
    def _sc_encoder(x_hbm, embp_hbm, emb2_hbm, out_hbm, t0, t1, t2, t3, t4,
                    t5, t6, t7, t8, raw0, raw1, o0, o1, il0, il1,
                    idx_sem, tab_sem, gsem0, gsem1, osem0, osem1):
        _sc_body(x_hbm, embp_hbm, emb2_hbm, out_hbm,
                 (t0, t1, t2, t3, t4, t5, t6, t7, t8),
                 (raw0, raw1), (o0, o1), (il0, il1),
                 idx_sem, tab_sem, (gsem0, gsem1), (osem0, osem1))

    return _sc_encoder


def kernel(x, emb):
    emb_bf = emb.astype(jnp.bfloat16)
    packed = jax.lax.bitcast_convert_type(
        jnp.stack([emb_bf[..., :HW], emb_bf[..., HW:]], axis=-1),
        jnp.int32).reshape(F * V * HW)
    flat = emb.reshape(F * V, H)
    return _build_encoder()(x.astype(jnp.int32), packed, flat)


# speedup vs baseline: 1.1371x; 1.1371x over previous
import functools

import jax
import jax.numpy as jnp
from jax import lax
from jax.experimental import pallas as pl
from jax.experimental.pallas import tpu as pltpu
from jax.experimental.pallas import tpu_sc as plsc

N = 100000
F = 9
V = 100
H = 128
HW = H // 2
L = 16
SC_R = 128
RT = 88
MS = SC_R - RT
SMS = ((MS + L - 1) // L) * L
NW = 32
NSC = (N + SC_R - 1) // SC_R


def _sc_body(x_hbm, embp_hbm, emb2_hbm, out_hbm, tables, raws, outs, idxs,
             idx_sem, tab_sem, gsems, osems):
    cid = lax.axis_index("c")
    sid = lax.axis_index("s")
    wid = sid * 2 + cid
    nj = (NSC - wid + NW - 1) // NW

    lanes = lax.broadcasted_iota(jnp.int32, (L,), 0)
    cols = [jnp.int32(cc * L) + lanes for cc in range(HW // L)]
    zeros = jnp.zeros((L,), jnp.float32)
    himask = jnp.full((L,), -65536, jnp.int32)

    def sbase(j):
        chunk = wid + j * NW
        return jnp.minimum(chunk * SC_R, N - SC_R)

    def fire_raw(j, b):
        return pltpu.async_copy(x_hbm.at[pl.ds(sbase(j), SC_R)], raws[b],
                                idx_sem)

    def wait_gathers(b):
        for _ in range(F):
            pltpu.make_async_copy(emb2_hbm.at[idxs[b].at[pl.ds(0, MS)]],
                                  outs[b].at[pl.ds(RT, MS)], gsems[b]).wait()

    def fire_out(j, b):
        return pltpu.async_copy(outs[b], out_hbm.at[pl.ds(sbase(j), SC_R)],
                                osems[b])

    def wait_out(b):
        pltpu.make_async_copy(outs[b], out_hbm.at[pl.ds(0, SC_R)],
                              osems[b]).wait()

    @pl.when(nj > 0)
    def _():
        fire_raw(0, 0)
    tcps = [pltpu.async_copy(embp_hbm.at[pl.ds(i * V * HW, V * HW)],
                             tables[i], tab_sem) for i in range(F)]
    for cp in tcps:
        cp.wait()

    def do_super(j, b):
        pltpu.make_async_copy(x_hbm.at[pl.ds(0, SC_R)], raws[b],
                              idx_sem).wait()

        @pl.when(j + 1 < nj)
        def _():
            fire_raw(j + 1, 1 - b)

        @pl.when(j >= 1)
        def _():
            wait_gathers(1 - b)
            fire_out(j - 1, 1 - b)

        @pl.when(j >= 2)
        def _():
            wait_out(b)

        for k in range(SMS // L):
            srows = jnp.minimum(jnp.int32(k * L) + lanes, MS - 1) + RT
            for i in range(F):
                vals = plsc.load_gather(raws[b],
                                        [srows, jnp.full((L,), i, jnp.int32)])
                idxs[b][pl.ds(i * SMS + k * L, L)] = vals + (i * V)

        def zrow(r, carry):
            for m in range(H // L):
                outs[b][r, pl.ds(m * L, L)] = zeros
            return carry

        lax.fori_loop(RT, SC_R, zrow, 0)

        for i in range(F):
            pltpu.async_copy(
                emb2_hbm.at[idxs[b].at[pl.ds(i * SMS, MS)]],
                outs[b].at[pl.ds(RT, MS)], gsems[b], add=True)

        def one_row(r):
            bases = []
            for i in range(F):
                bidx = plsc.load_gather(raws[b],
                                        [jnp.full((L,), r, jnp.int32),
                                         jnp.full((L,), i, jnp.int32)])
                bases.append(lax.shift_left(bidx, 6))
            for cc in range(HW // L):
                ws = [plsc.bitcast(
                          plsc.load_gather(tables[i], [bases[i] + cols[cc]]),
                          jnp.bfloat16)
                      for i in range(F)]
                while len(ws) > 1:
                    ws = [a + b2 for a, b2 in zip(ws[::2], ws[1::2])] \
                        + ([ws[-1]] if len(ws) % 2 else [])
                ai = plsc.bitcast(ws[0], jnp.int32)
                outs[b][r, pl.ds(cc * L, L)] = plsc.bitcast(
                    lax.shift_left(ai, 16), jnp.float32)
                outs[b][r, pl.ds(HW + cc * L, L)] = plsc.bitcast(
                    lax.bitwise_and(ai, himask), jnp.float32)

        def row_step(rr, carry):
            one_row(rr * 2)
            one_row(rr * 2 + 1)
            return carry

        lax.fori_loop(0, RT // 2, row_step, 0)

    def pair_step(jj, carry):
        j0 = jj * 2

        @pl.when(j0 < nj)
        def _():
            do_super(j0, 0)

        @pl.when(j0 + 1 < nj)
        def _():
            do_super(j0 + 1, 1)

        return carry

    lax.fori_loop(0, (nj + 1) // 2, pair_step, 0)

    for b in range(2):
        @pl.when((nj > 0) & ((nj - 1) % 2 == b))
        def _():
            wait_gathers(b)
            fire_out(nj - 1, b)
            wait_out(b)

        @pl.when((nj > 1) & ((nj - 2) % 2 == b))
        def _():
            wait_out(b)


@functools.lru_cache(maxsize=1)
def _build_encoder():
    @functools.partial(
        pl.kernel,
        out_type=jax.ShapeDtypeStruct((N, H), jnp.float32),
        mesh=plsc.VectorSubcoreMesh(core_axis_name="c", subcore_axis_name="s"),
        compiler_params=pltpu.CompilerParams(needs_layout_passes=False),
        scratch_types=(
            [pltpu.VMEM((V * HW,), jnp.int32) for _ in range(F)]
            + [pltpu.VMEM((SC_R, F), jnp.int32) for _ in range(2)]
            + [pltpu.VMEM((SC_R, H), jnp.float32) for _ in range(2)]
            + [pltpu.VMEM((F * SMS,), jnp.int32) for _ in range(2)]
            + [pltpu.SemaphoreType.DMA] * 6
        ),
    )
    def _sc_encoder(x_hbm, embp_hbm, emb2_hbm, out_hbm, t0, t1, t2, t3, t4,
                    t5, t6, t7, t8, raw0, raw1, o0, o1, il0, il1,
                    idx_sem, tab_sem, gsem0, gsem1, osem0, osem1):
        _sc_body(x_hbm, embp_hbm, emb2_hbm, out_hbm,
                 (t0, t1, t2, t3, t4, t5, t6, t7, t8),
                 (raw0, raw1), (o0, o1), (il0, il1),
                 idx_sem, tab_sem, (gsem0, gsem1), (osem0, osem1))

    return _sc_encoder


def kernel(x, emb):
    emb_bf = emb.astype(jnp.bfloat16)
    packed = jax.lax.bitcast_convert_type(
        jnp.stack([emb_bf[..., :HW], emb_bf[..., HW:]], axis=-1),
        jnp.int32).reshape(F * V * HW)
    flat = emb.reshape(F * V, H)
    return _build_encoder()(x.astype(jnp.int32), packed, flat)
